# baseline (speedup 1.0000x reference)
import dataclasses
import functools

import jax
import jax.numpy as jnp
from jax import lax
from jax.experimental import pallas as pl
from jax.experimental.pallas import tpu as pltpu
from jax.experimental.pallas import tpu_sc as plsc

NC = 2
NS = 16
L = 16
NW = NC * NS

D = 64
CHUNK = 128
NG = CHUNK // L


def _transe_kernel(B, NE, NR, b_per_w, n_chunks):
    mesh = plsc.VectorSubcoreMesh(core_axis_name="c", subcore_axis_name="s")
    cp = pltpu.CompilerParams()
    if "needs_layout_passes" in pltpu.CompilerParams.__dataclass_fields__:
        cp = dataclasses.replace(cp, needs_layout_passes=False)

    @functools.partial(
        pl.kernel,
        mesh=mesh,
        compiler_params=cp,
        out_type=jax.ShapeDtypeStruct((B,), jnp.float32),
        scratch_types=[
            pltpu.VMEM((b_per_w,), jnp.int32),
            pltpu.VMEM((b_per_w,), jnp.int32),
            pltpu.VMEM((b_per_w,), jnp.int32),
            pltpu.VMEM((D, CHUNK), jnp.int32),
            pltpu.VMEM((D, CHUNK), jnp.int32),
            pltpu.VMEM((D, CHUNK), jnp.int32),
            pltpu.VMEM((D, CHUNK), jnp.float32),
            pltpu.VMEM((D, CHUNK), jnp.float32),
            pltpu.VMEM((D, CHUNK), jnp.float32),
            pltpu.VMEM((CHUNK,), jnp.float32),
            pltpu.SemaphoreType.DMA,
            pltpu.SemaphoreType.DMA,
            pltpu.SemaphoreType.DMA,
        ],
    )
    def k(heads_hbm, rels_hbm, tails_hbm, entf_hbm, relf_hbm, out_hbm,
          hidx_v, ridx_v, tidx_v, hjx, rjx, tjx, hbuf, rbuf, tbuf, out_v,
          semh, semr, semt):
        wid = lax.axis_index("s") * NC + lax.axis_index("c")
        base = wid * b_per_w
        pltpu.sync_copy(heads_hbm.at[pl.ds(base, b_per_w)], hidx_v)
        pltpu.sync_copy(rels_hbm.at[pl.ds(base, b_per_w)], ridx_v)
        pltpu.sync_copy(tails_hbm.at[pl.ds(base, b_per_w)], tidx_v)

        @pl.loop(0, n_chunks)
        def _(c):
            off = c * CHUNK
            hv = [hidx_v[pl.ds(off + g * L, L)] for g in range(NG)]
            rv = [ridx_v[pl.ds(off + g * L, L)] for g in range(NG)]
            tv = [tidx_v[pl.ds(off + g * L, L)] for g in range(NG)]

            @pl.loop(0, D)
            def _(j):
                he = j * NE
                re = j * NR
                for g in range(NG):
                    sl = pl.ds(g * L, L)
                    hjx[j, sl] = hv[g] + he
                    rjx[j, sl] = rv[g] + re
                    tjx[j, sl] = tv[g] + he

            @pl.loop(0, D)
            def _(j):
                pltpu.async_copy(entf_hbm.at[hjx.at[j]], hbuf.at[j], semh)
                pltpu.async_copy(relf_hbm.at[rjx.at[j]], rbuf.at[j], semr)
                pltpu.async_copy(entf_hbm.at[tjx.at[j]], tbuf.at[j], semt)

            @pl.loop(0, D)
            def _(j):
                pltpu.make_async_copy(
                    entf_hbm.at[hjx.at[0]], hbuf.at[0], semh).wait()
                pltpu.make_async_copy(
                    relf_hbm.at[rjx.at[0]], rbuf.at[0], semr).wait()
                pltpu.make_async_copy(
                    entf_hbm.at[tjx.at[0]], tbuf.at[0], semt).wait()

            for g in range(NG):
                sl = pl.ds(g * L, L)
                acc = None
                for j in range(D):
                    d = hbuf[j, sl] + rbuf[j, sl] - tbuf[j, sl]
                    acc = d * d if acc is None else acc + d * d
                y = acc
                i = jnp.int32(0x5F3759DF) - lax.shift_right_logical(
                    plsc.bitcast(y, jnp.int32), 1)
                rs = plsc.bitcast(i, jnp.float32)
                nh = y * jnp.float32(-0.5)
                rs = rs * (jnp.float32(1.5) + nh * rs * rs)
                rs = rs * (jnp.float32(1.5) + nh * rs * rs)
                out_v[sl] = jnp.float32(0.0) - y * rs

            pltpu.sync_copy(out_v, out_hbm.at[pl.ds(base + off, CHUNK)])

    return k


def kernel(heads, rels, tails, ent_embs, rel_embs):
    B = heads.shape[0]
    NE, Dm = ent_embs.shape
    NR = rel_embs.shape[0]
    b_per_w = B // NW
    n_chunks = b_per_w // CHUNK
    entf = ent_embs.T.reshape(-1)
    relf = rel_embs.T.reshape(-1)
    k = _transe_kernel(B, NE, NR, b_per_w, n_chunks)
    return k(heads.astype(jnp.int32), rels.astype(jnp.int32),
             tails.astype(jnp.int32), entf, relf)


# speedup vs baseline: 9.0330x; 9.0330x over previous
import dataclasses
import functools

import jax
import jax.numpy as jnp
from jax import lax
from jax.experimental import pallas as pl
from jax.experimental.pallas import tpu as pltpu
from jax.experimental.pallas import tpu_sc as plsc

NC = 2
NS = 16
L = 16
NW = NC * NS

D = 64
DP = 128
CHUNK = 128
TPAD = 24


def _transe_kernel(B, b_per_w, n_chunks):
    mesh = plsc.VectorSubcoreMesh(core_axis_name="c", subcore_axis_name="s")
    cp = pltpu.CompilerParams(use_tc_tiling_on_sc=False)
    if "needs_layout_passes" in pltpu.CompilerParams.__dataclass_fields__:
        cp = dataclasses.replace(cp, needs_layout_passes=False)

    @functools.partial(
        pl.kernel,
        mesh=mesh,
        compiler_params=cp,
        out_type=jax.ShapeDtypeStruct((B,), jnp.float32),
        scratch_types=[
            pltpu.VMEM((b_per_w,), jnp.int32),
            pltpu.VMEM((b_per_w,), jnp.int32),
            pltpu.VMEM((b_per_w,), jnp.int32),
            pltpu.VMEM((CHUNK, DP), jnp.float32),
            pltpu.VMEM((CHUNK, DP), jnp.float32),
            pltpu.VMEM((CHUNK, DP), jnp.float32),
            pltpu.VMEM((CHUNK,), jnp.float32),
            pltpu.VMEM((L * TPAD,), jnp.float32),
            pltpu.SemaphoreType.DMA,
            pltpu.SemaphoreType.DMA,
            pltpu.SemaphoreType.DMA,
        ],
    )
    def k(heads_hbm, rels_hbm, tails_hbm, ent_hbm, rel_hbm, out_hbm,
          hidx_v, ridx_v, tidx_v, hrows, rrows, trows, out_v, tsc,
          semh, semr, semt):
        wid = lax.axis_index("s") * NC + lax.axis_index("c")
        base = wid * b_per_w
        pltpu.sync_copy(heads_hbm.at[pl.ds(base, b_per_w)], hidx_v)
        pltpu.sync_copy(rels_hbm.at[pl.ds(base, b_per_w)], ridx_v)
        pltpu.sync_copy(tails_hbm.at[pl.ds(base, b_per_w)], tidx_v)

        tbase = lax.iota(jnp.int32, L) * TPAD

        @pl.loop(0, n_chunks)
        def _(c):
            off = c * CHUNK
            ch = pltpu.async_copy(
                ent_hbm.at[hidx_v.at[pl.ds(off, CHUNK)]], hrows, semh)
            cr = pltpu.async_copy(
                rel_hbm.at[ridx_v.at[pl.ds(off, CHUNK)]], rrows, semr)
            ct = pltpu.async_copy(
                ent_hbm.at[tidx_v.at[pl.ds(off, CHUNK)]], trows, semt)
            ch.wait()
            cr.wait()
            ct.wait()

            @pl.loop(0, CHUNK // L)
            def _(blk):
                row0 = blk * L
                for r_ in range(L):
                    row = row0 + r_
                    acc = None
                    for q in range(D // L):
                        sl = pl.ds(q * L, L)
                        d = hrows[row, sl] + rrows[row, sl] - trows[row, sl]
                        acc = d * d if acc is None else acc + d * d
                    tsc[pl.ds(r_ * TPAD, L)] = acc
                y = plsc.load_gather(tsc, [tbase])
                for j in range(1, L):
                    y = y + plsc.load_gather(tsc, [tbase + j])
                i = jnp.int32(0x5F3759DF) - lax.shift_right_logical(
                    plsc.bitcast(y, jnp.int32), 1)
                rs = plsc.bitcast(i, jnp.float32)
                nh = y * jnp.float32(-0.5)
                rs = rs * (jnp.float32(1.5) + nh * rs * rs)
                rs = rs * (jnp.float32(1.5) + nh * rs * rs)
                out_v[pl.ds(row0, L)] = jnp.float32(0.0) - y * rs

            pltpu.sync_copy(out_v, out_hbm.at[pl.ds(base + off, CHUNK)])

    return k


def kernel(heads, rels, tails, ent_embs, rel_embs):
    B = heads.shape[0]
    b_per_w = B // NW
    n_chunks = b_per_w // CHUNK
    entp = jnp.pad(ent_embs, ((0, 0), (0, DP - ent_embs.shape[1])))
    relp = jnp.pad(rel_embs, ((0, 0), (0, DP - rel_embs.shape[1])))
    k = _transe_kernel(B, b_per_w, n_chunks)
    return k(heads.astype(jnp.int32), rels.astype(jnp.int32),
             tails.astype(jnp.int32), entp, relp)
